# baseline (speedup 1.0000x reference)
import functools

import jax
import jax.numpy as jnp
from jax import lax
from jax.experimental import pallas as pl
from jax.experimental.pallas import tpu as pltpu
from jax.experimental.pallas import tpu_sc as plsc

B = 16384
Z_DIM = 64
NC = 2
NS = 16
NW = NC * NS
BPW = B // NW
CHUNK = 128
NCHUNK = BPW // CHUNK
L = 16


@functools.partial(
    pl.kernel,
    mesh=plsc.VectorSubcoreMesh(core_axis_name="c", subcore_axis_name="s"),
    out_type=(
        jax.ShapeDtypeStruct((Z_DIM, B), jnp.float32),
        jax.ShapeDtypeStruct((B,), jnp.float32),
        jax.ShapeDtypeStruct((B,), jnp.float32),
    ),
    scratch_types=[
        pltpu.VMEM((BPW,), jnp.int32),
        pltpu.VMEM((BPW, Z_DIM), jnp.float32),
        pltpu.VMEM((Z_DIM, BPW), jnp.float32),
        pltpu.VMEM((BPW,), jnp.float32),
        pltpu.VMEM((BPW,), jnp.float32),
        pltpu.SemaphoreType.DMA,
        pltpu.SemaphoreType.DMA,
    ],
    compiler_params=pltpu.CompilerParams(
        use_tc_tiling_on_sc=True, needs_layout_passes=False),
)
def _sc_gather(idx_hbm, emb_hbm, inh_hbm, det_hbm,
               z_out, inh_out, det_out,
               idx_v, z_v, zt_v, inh_v, det_v, sem, row_sem):
    wid = lax.axis_index("s") * NC + lax.axis_index("c")
    base = wid * BPW
    pltpu.sync_copy(idx_hbm.at[pl.ds(base, BPW)], idx_v)
    copies = []
    for c in range(NCHUNK):
        sl = pl.ds(c * CHUNK, CHUNK)
        copies.append(pltpu.async_copy(inh_hbm.at[idx_v.at[sl]], inh_v.at[sl], sem))
        copies.append(pltpu.async_copy(det_hbm.at[idx_v.at[sl]], det_v.at[sl], sem))

    def body(g, carry):
        vec = idx_v[pl.ds(g * L, L)]
        for t in range(L):
            j = g * L + t
            pltpu.async_copy(
                emb_hbm.at[pl.ds(vec[t], 1)], z_v.at[pl.ds(j, 1)], row_sem)
        return carry

    lax.fori_loop(0, BPW // L, body, 0)
    pltpu.make_async_copy(emb_hbm.at[pl.ds(0, BPW)], z_v, row_sem).wait()
    lane = lax.iota(jnp.int32, L)
    for g in range(BPW // L):
        rows = lane + g * L
        for c in range(Z_DIM):
            cols = jnp.full((L,), c, jnp.int32)
            zt_v[c, pl.ds(g * L, L)] = plsc.load_gather(z_v, [rows, cols])
    for cp in copies:
        cp.wait()
    pltpu.sync_copy(zt_v, z_out.at[:, pl.ds(base, BPW)])
    pltpu.sync_copy(inh_v, inh_out.at[pl.ds(base, BPW)])
    pltpu.sync_copy(det_v, det_out.at[pl.ds(base, BPW)])


def kernel(piano_model, emb_w, inharm_w, detune_w):
    idx = piano_model.astype(jnp.int32)
    z_t, inh, det = _sc_gather(idx, emb_w,
                               inharm_w.reshape(-1), detune_w.reshape(-1))
    return (z_t.T[:, None, :],
            inh.reshape(B, 1, 1),
            det.reshape(B, 1, 1))


# speedup vs baseline: 1.3146x; 1.3146x over previous
import functools

import jax
import jax.numpy as jnp
from jax import lax
from jax.experimental import pallas as pl
from jax.experimental.pallas import tpu as pltpu
from jax.experimental.pallas import tpu_sc as plsc

B = 16384
Z_DIM = 64
NC = 2
NS = 16
NW = NC * NS
BPW = B // NW
CHUNK = 128
NCHUNK = BPW // CHUNK
L = 16


@functools.partial(
    pl.kernel,
    mesh=plsc.VectorSubcoreMesh(core_axis_name="c", subcore_axis_name="s"),
    out_type=(
        jax.ShapeDtypeStruct((B, Z_DIM), jnp.float32),
        jax.ShapeDtypeStruct((B,), jnp.float32),
        jax.ShapeDtypeStruct((B,), jnp.float32),
    ),
    scratch_types=[
        pltpu.VMEM((BPW,), jnp.int32),
        pltpu.VMEM((BPW, Z_DIM), jnp.float32),
        pltpu.VMEM((BPW,), jnp.float32),
        pltpu.VMEM((BPW,), jnp.float32),
        pltpu.SemaphoreType.DMA,
        pltpu.SemaphoreType.DMA,
    ],
    compiler_params=pltpu.CompilerParams(use_tc_tiling_on_sc=True),
)
def _sc_gather(idx_hbm, emb_hbm, inh_hbm, det_hbm,
               z_out, inh_out, det_out,
               idx_v, z_v, inh_v, det_v, sem, row_sem):
    wid = lax.axis_index("s") * NC + lax.axis_index("c")
    base = wid * BPW
    pltpu.sync_copy(idx_hbm.at[pl.ds(base, BPW)], idx_v)
    copies = []
    for c in range(NCHUNK):
        sl = pl.ds(c * CHUNK, CHUNK)
        copies.append(pltpu.async_copy(inh_hbm.at[idx_v.at[sl]], inh_v.at[sl], sem))
        copies.append(pltpu.async_copy(det_hbm.at[idx_v.at[sl]], det_v.at[sl], sem))

    def body(g, carry):
        vec = idx_v[pl.ds(g * L, L)]
        for t in range(L):
            j = g * L + t
            pltpu.async_copy(
                emb_hbm.at[pl.ds(vec[t], 1)], z_v.at[pl.ds(j, 1)], row_sem)
        return carry

    lax.fori_loop(0, BPW // L, body, 0)
    pltpu.make_async_copy(emb_hbm.at[pl.ds(0, BPW)], z_v, row_sem).wait()
    for cp in copies:
        cp.wait()
    pltpu.sync_copy(z_v, z_out.at[pl.ds(base, BPW)])
    pltpu.sync_copy(inh_v, inh_out.at[pl.ds(base, BPW)])
    pltpu.sync_copy(det_v, det_out.at[pl.ds(base, BPW)])


def kernel(piano_model, emb_w, inharm_w, detune_w):
    idx = piano_model.astype(jnp.int32)
    z, inh, det = _sc_gather(idx, emb_w,
                             inharm_w.reshape(-1), detune_w.reshape(-1))
    return (z[:, None, :],
            inh.reshape(B, 1, 1),
            det.reshape(B, 1, 1))
